# baseline (device time: 26786 ns/iter reference)
import jax
import jax.numpy as jnp
from jax import lax
from jax.experimental import pallas as pl
from jax.experimental.pallas import tpu as pltpu

N_DEV = 4


def kernel(x, w_mat):
    k_dim, m_blk = x.shape
    _, n = w_mat.shape

    def body(x_hbm, w_hbm, out_ref,
             x_vmem, send_buf, comm_ref, w_vmem,
             x_sems, send_sems, recv_sems, w_sems):
        my = lax.axis_index("i")

        XORDER = (1, 3, 2, 0)
        x_copies = []
        for s, h in enumerate(XORDER):
            j = lax.rem(my + h, N_DEV)
            cp = pltpu.make_async_copy(
                x_hbm.at[pl.ds(j * m_blk, m_blk), :],
                x_vmem.at[s],
                x_sems.at[s],
            )
            cp.start()
            x_copies.append(cp)

        barrier = pltpu.get_barrier_semaphore()
        for h in range(1, N_DEV):
            peer = lax.rem(my + h, N_DEV)
            pl.semaphore_signal(
                barrier, inc=1,
                device_id=(peer,), device_id_type=pl.DeviceIdType.MESH,
            )
        pl.semaphore_wait(barrier, N_DEV - 1)

        sends = []
        for s, h in enumerate(XORDER[:3]):
            peer = lax.rem(my + h, N_DEV)
            x_copies[s].wait()
            send_buf[s] = x_vmem[s].astype(jnp.bfloat16)
            rdma = pltpu.make_async_remote_copy(
                src_ref=send_buf.at[s],
                dst_ref=comm_ref.at[my],
                send_sem=send_sems.at[s],
                recv_sem=recv_sems.at[my],
                device_id=(peer,),
                device_id_type=pl.DeviceIdType.MESH,
            )
            rdma.start()
            sends.append(rdma)

        WORDER = (0, 1, 3, 2)
        w_copies = []
        for s, h in enumerate(WORDER):
            j = lax.rem(my + h, N_DEV)
            cp = pltpu.make_async_copy(
                w_hbm.at[pl.ds(j * m_blk, m_blk), :],
                w_vmem.at[s],
                w_sems.at[s],
            )
            cp.start()
            w_copies.append(cp)

        x_copies[3].wait()
        w_copies[0].wait()
        acc = jnp.dot(
            x_vmem[3],
            w_vmem[0],
            preferred_element_type=jnp.float32,
        )

        for s, h in ((1, 1), (2, 3), (3, 2)):
            j = lax.rem(my + h, N_DEV)
            recv = pltpu.make_async_remote_copy(
                src_ref=send_buf.at[0],
                dst_ref=comm_ref.at[j],
                send_sem=send_sems.at[0],
                recv_sem=recv_sems.at[j],
                device_id=(j,),
                device_id_type=pl.DeviceIdType.MESH,
            )
            recv.wait_recv()
            w_copies[s].wait()
            acc = acc + jnp.dot(
                comm_ref[j].astype(jnp.float32),
                w_vmem[s],
                preferred_element_type=jnp.float32,
            )

        out_ref[:, :] = acc * jax.nn.sigmoid(acc)

        for rdma in sends:
            rdma.wait_send()

    return pl.pallas_call(
        body,
        out_shape=jax.ShapeDtypeStruct((m_blk, n), jnp.float32),
        in_specs=[
            pl.BlockSpec(memory_space=pl.ANY),
            pl.BlockSpec(memory_space=pl.ANY),
        ],
        out_specs=pl.BlockSpec(memory_space=pltpu.VMEM),
        scratch_shapes=[
            pltpu.VMEM((N_DEV, m_blk, m_blk), jnp.float32),
            pltpu.VMEM((N_DEV - 1, m_blk, m_blk), jnp.bfloat16),
            pltpu.VMEM((N_DEV, m_blk, m_blk), jnp.bfloat16),
            pltpu.VMEM((N_DEV, m_blk, n), jnp.float32),
            pltpu.SemaphoreType.DMA((N_DEV,)),
            pltpu.SemaphoreType.DMA((N_DEV - 1,)),
            pltpu.SemaphoreType.DMA((N_DEV,)),
            pltpu.SemaphoreType.DMA((N_DEV,)),
        ],
        compiler_params=pltpu.CompilerParams(
            collective_id=0,
            vmem_limit_bytes=64 * 1024 * 1024,
        ),
    )(x, w_mat)


# device time: 22218 ns/iter; 1.2056x vs baseline; 1.2056x over previous
import jax
import jax.numpy as jnp
from jax import lax
from jax.experimental import pallas as pl
from jax.experimental.pallas import tpu as pltpu

N_DEV = 4


def kernel(x, w_mat):
    k_dim, m_blk = x.shape
    _, n = w_mat.shape

    def body(x_ref, w_hbm, out_ref,
             send_buf, comm_ref, w_vmem,
             send_sems, recv_sems, w_sems):
        my = lax.axis_index("i")

        barrier = pltpu.get_barrier_semaphore()
        for h in range(1, N_DEV):
            peer = lax.rem(my + h, N_DEV)
            pl.semaphore_signal(
                barrier, inc=1,
                device_id=(peer,), device_id_type=pl.DeviceIdType.MESH,
            )

        for s, h in enumerate((1, 3, 2)):
            peer = lax.rem(my + h, N_DEV)
            send_buf[s] = x_ref[pl.ds(peer * m_blk, m_blk), :].astype(
                jnp.bfloat16
            )

        pl.semaphore_wait(barrier, N_DEV - 1)
        sends = []
        for s, h in enumerate((1, 3, 2)):
            peer = lax.rem(my + h, N_DEV)
            rdma = pltpu.make_async_remote_copy(
                src_ref=send_buf.at[s],
                dst_ref=comm_ref.at[my],
                send_sem=send_sems.at[s],
                recv_sem=recv_sems.at[my],
                device_id=(peer,),
                device_id_type=pl.DeviceIdType.MESH,
            )
            rdma.start()
            sends.append(rdma)

        WORDER = (0, 1, 3, 2)
        w_copies = []
        for s, h in enumerate(WORDER):
            j = lax.rem(my + h, N_DEV)
            cp = pltpu.make_async_copy(
                w_hbm.at[pl.ds(j * m_blk, m_blk), :],
                w_vmem.at[s],
                w_sems.at[s],
            )
            cp.start()
            w_copies.append(cp)

        w_copies[0].wait()
        out_ref[:, :] = jnp.dot(
            x_ref[pl.ds(my * m_blk, m_blk), :],
            w_vmem[0],
            preferred_element_type=jnp.float32,
        )

        for s, h in ((1, 1), (2, 3), (3, 2)):
            j = lax.rem(my + h, N_DEV)
            recv = pltpu.make_async_remote_copy(
                src_ref=send_buf.at[0],
                dst_ref=comm_ref.at[j],
                send_sem=send_sems.at[0],
                recv_sem=recv_sems.at[j],
                device_id=(j,),
                device_id_type=pl.DeviceIdType.MESH,
            )
            recv.wait_recv()
            w_copies[s].wait()
            out_ref[:, :] += jnp.dot(
                comm_ref[j].astype(jnp.float32),
                w_vmem[s],
                preferred_element_type=jnp.float32,
            )

        acc = out_ref[:, :]
        out_ref[:, :] = acc * jax.nn.sigmoid(acc)

        for rdma in sends:
            rdma.wait_send()

    return pl.pallas_call(
        body,
        out_shape=jax.ShapeDtypeStruct((m_blk, n), jnp.float32),
        in_specs=[
            pl.BlockSpec(memory_space=pltpu.VMEM),
            pl.BlockSpec(memory_space=pl.ANY),
        ],
        out_specs=pl.BlockSpec(memory_space=pltpu.VMEM),
        scratch_shapes=[
            pltpu.VMEM((N_DEV - 1, m_blk, m_blk), jnp.bfloat16),
            pltpu.VMEM((N_DEV, m_blk, m_blk), jnp.bfloat16),
            pltpu.VMEM((N_DEV, m_blk, n), jnp.float32),
            pltpu.SemaphoreType.DMA((N_DEV - 1,)),
            pltpu.SemaphoreType.DMA((N_DEV,)),
            pltpu.SemaphoreType.DMA((N_DEV,)),
        ],
        compiler_params=pltpu.CompilerParams(collective_id=0),
    )(x, w_mat)
